# baseline (speedup 1.0000x reference)
import functools

import jax
import jax.numpy as jnp
from jax import lax
from jax.experimental import pallas as pl
from jax.experimental.pallas import tpu as pltpu
from jax.experimental.pallas import tpu_sc as plsc

N = 10000
E = 320000
D = 128
H = 128
G = 128

NC = 2
NS = 16
NW = NC * NS

CH = 128
TOT_CHUNKS = E // CH
NB = 3
NLOOP = 27
ACC_ROWS = N
RPT_A = 624
RPT_B = N - RPT_A * (NS - 1)
EPS = 1e-5

_HIGH = jax.lax.Precision.HIGHEST


@functools.cache
def _agg_kernel():
    mesh = plsc.VectorSubcoreMesh(core_axis_name="c", subcore_axis_name="s")

    @functools.partial(
        pl.kernel,
        mesh=mesh,
        out_type=jax.ShapeDtypeStruct((NC * ACC_ROWS, H), jnp.float32),
        scratch_types=[
            pltpu.VMEM((1, CH), jnp.int32),
            pltpu.VMEM((1, CH), jnp.int32),
            pltpu.VMEM((1, CH), jnp.int32),
            pltpu.VMEM((1, CH), jnp.int32),
            pltpu.VMEM((1, CH), jnp.int32),
            pltpu.VMEM((1, CH), jnp.int32),
            pltpu.VMEM((CH, H), jnp.float32),
            pltpu.VMEM((CH, H), jnp.float32),
            pltpu.VMEM((CH, H), jnp.float32),
            pltpu.VMEM_SHARED((ACC_ROWS, H), jnp.float32),
            pltpu.SemaphoreType.DMA,
            pltpu.SemaphoreType.DMA,
            pltpu.SemaphoreType.DMA,
            pltpu.SemaphoreType.DMA,
            pltpu.SemaphoreType.DMA,
            pltpu.SemaphoreType.DMA,
            pltpu.SemaphoreType.DMA,
            pltpu.SemaphoreType.DMA,
            pltpu.SemaphoreType.DMA,
        ],
    )
    def agg(hs_hbm, src_hbm, dst_hbm, zeros_hbm, out_hbm,
            sb0, sb1, sb2, db0, db1, db2, r0, r1, r2, acc_sh,
            g0, g1, g2, s0, s1, s2, i0, i1, i2):
        c = lax.axis_index("c")
        s = lax.axis_index("s")
        wid = s * NC + c
        cbase = (wid * TOT_CHUNKS) // NW
        cend = ((wid + 1) * TOT_CHUNKS) // NW
        sb = (sb0, sb1, sb2)
        db = (db0, db1, db2)
        rows = (r0, r1, r2)
        gsem = (g0, g1, g2)
        ssem = (s0, s1, s2)
        isem = (i0, i1, i2)

        for b in range(NB):
            pltpu.async_copy(src_hbm.at[cbase + b], sb[b], isem[b])
            pltpu.async_copy(dst_hbm.at[cbase + b], db[b], isem[b])
        for b in range(NB):
            pltpu.make_async_copy(src_hbm.at[0], sb[b], isem[b]).wait()
            pltpu.async_copy(hs_hbm.at[sb[b].at[0]], rows[b], gsem[b])

        @pl.when(s < NS - 1)
        def _():
            pltpu.sync_copy(zeros_hbm.at[pl.ds(0, RPT_A)],
                            acc_sh.at[pl.ds(s * RPT_A, RPT_A)])

        @pl.when(s == NS - 1)
        def _():
            pltpu.sync_copy(zeros_hbm,
                            acc_sh.at[pl.ds((NS - 1) * RPT_A, RPT_B)])

        plsc.subcore_barrier()

        def step(i, carry):
            for b in range(NB):
                gj = cbase + NB * i + b
                have = gj < cend
                nxt = gj + NB < cend

                @pl.when(have)
                def _():
                    pltpu.make_async_copy(hs_hbm.at[pl.ds(0, CH)],
                                          rows[b], gsem[b]).wait()
                    pltpu.make_async_copy(dst_hbm.at[0], db[b],
                                          isem[b]).wait()
                    pltpu.async_copy(rows[b], acc_sh.at[db[b].at[0]],
                                     ssem[b], add=True)

                @pl.when(nxt)
                def _():
                    pltpu.async_copy(src_hbm.at[gj + NB], sb[b], isem[b])

                @pl.when(have)
                def _():
                    pltpu.make_async_copy(rows[b], acc_sh.at[pl.ds(0, CH)],
                                          ssem[b]).wait()

                @pl.when(nxt)
                def _():
                    pltpu.async_copy(dst_hbm.at[gj + NB], db[b], isem[b])
                    pltpu.make_async_copy(src_hbm.at[0], sb[b],
                                          isem[b]).wait()
                    pltpu.async_copy(hs_hbm.at[sb[b].at[0]], rows[b], gsem[b])
            return carry

        lax.fori_loop(0, NLOOP, step, 0)
        plsc.subcore_barrier()

        @pl.when(s < NS - 1)
        def _():
            pltpu.sync_copy(acc_sh.at[pl.ds(s * RPT_A, RPT_A)],
                            out_hbm.at[pl.ds(c * ACC_ROWS + s * RPT_A, RPT_A)])

        @pl.when(s == NS - 1)
        def _():
            pltpu.sync_copy(
                acc_sh.at[pl.ds((NS - 1) * RPT_A, RPT_B)],
                out_hbm.at[pl.ds(c * ACC_ROWS + (NS - 1) * RPT_A, RPT_B)])

    return agg


CHE = 1280
NHC = E // CHE
HI = 80


def _deg_body(dst_ref, dis_ref):
    def body(j, acc):
        d2 = dst_ref[pl.ds(j, 1), :]
        hi2 = lax.shift_right_logical(d2, 7)
        lo2 = lax.bitwise_and(d2, 127)
        ohhi = (lax.broadcasted_iota(jnp.int32, (HI, CHE), 0) == hi2
                ).astype(jnp.float32)
        ohloT = (lax.broadcasted_iota(jnp.int32, (128, CHE), 0) == lo2
                 ).astype(jnp.float32)
        part = lax.dot_general(ohhi, ohloT, (((1,), (1,)), ((), ())),
                               preferred_element_type=jnp.float32)
        return acc + part

    deg = lax.fori_loop(0, NHC, body, jnp.zeros((HI, 128), jnp.float32))
    dis_ref[...] = lax.rsqrt(deg + 1.0)


_deg_call = pl.pallas_call(
    _deg_body,
    out_shape=jax.ShapeDtypeStruct((HI, 128), jnp.float32),
)


def _pre_body(x_ref, w_ref, dis_ref, hs_ref):
    h = lax.dot_general(x_ref[...], w_ref[...], (((1,), (0,)), ((), ())),
                        precision=_HIGH, preferred_element_type=jnp.float32)
    hs_ref[...] = h * dis_ref[...]


_pre_call = pl.pallas_call(
    _pre_body,
    out_shape=jax.ShapeDtypeStruct((N, H), jnp.float32),
)


def _mid_body(p_ref, hs_ref, dis_ref, b_ref, g_ref, be_ref, w_ref, out_ref):
    dis = dis_ref[...]
    agg = p_ref[0:N, :] + p_ref[ACC_ROWS:ACC_ROWS + N, :] + hs_ref[0:N, :]
    conv = agg * dis + b_ref[...]
    m = jnp.mean(conv, axis=0, keepdims=True)
    v = jnp.mean((conv - m) ** 2, axis=0, keepdims=True)
    y = (conv - m) * lax.rsqrt(v + EPS) * g_ref[...] + be_ref[...]
    y = jnp.maximum(y, 0.0)
    hn = lax.dot_general(y, w_ref[...], (((1,), (0,)), ((), ())),
                         precision=_HIGH, preferred_element_type=jnp.float32)
    out_ref[...] = hn * dis


_mid_call = pl.pallas_call(
    _mid_body,
    out_shape=jax.ShapeDtypeStruct((N, H), jnp.float32),
)


def _post_body(p_ref, hs_ref, dis_ref, b_ref, mask_ref, batch_ref,
               final_ref, out_ref):
    dis = dis_ref[...]
    agg = p_ref[0:N, :] + p_ref[ACC_ROWS:ACC_ROWS + N, :] + hs_ref[0:N, :]
    out = agg * dis + b_ref[...]
    pos = out * mask_ref[...]
    gids = lax.broadcasted_iota(jnp.int32, (G, N), 0)
    oh = (batch_ref[...] == gids).astype(jnp.float32)
    sums = lax.dot_general(oh, pos, (((1,), (0,)), ((), ())),
                           precision=_HIGH, preferred_element_type=jnp.float32)
    cnt = jnp.sum(oh, axis=1, keepdims=True)
    final_ref[...] = sums / jnp.maximum(cnt, 1.0)
    out_ref[...] = out


_post_call = pl.pallas_call(
    _post_body,
    out_shape=(jax.ShapeDtypeStruct((G, H), jnp.float32),
               jax.ShapeDtypeStruct((N, H), jnp.float32)),
)


def kernel(feature_matrix, edge_index, positive_doc_mask, batch,
           W1, b1, g1, be1, W2, b2, g2, be2, W3, b3):
    src = edge_index[0]
    dst = edge_index[1]
    src3d = src.reshape(TOT_CHUNKS, 1, CH)
    dst3d = dst.reshape(TOT_CHUNKS, 1, CH)

    zH = jnp.zeros((RPT_B, H), jnp.float32)
    maskf = positive_doc_mask.astype(jnp.float32)
    batch2 = batch.reshape(1, N)
    b1r, g1r, be1r = b1.reshape(1, H), g1.reshape(1, H), be1.reshape(1, H)
    b2r, g2r, be2r = b2.reshape(1, H), g2.reshape(1, H), be2.reshape(1, H)
    b3r = b3.reshape(1, H)

    agg = _agg_kernel()
    dis_grid = _deg_call(dst.reshape(NHC, CHE))
    dis = dis_grid.reshape(HI * 128, 1)[:N]
    hs1 = _pre_call(feature_matrix, W1, dis)
    p1 = agg(hs1, src3d, dst3d, zH)
    hs2 = _mid_call(p1, hs1, dis, b1r, g1r, be1r, W2)
    p2 = agg(hs2, src3d, dst3d, zH)
    hs3 = _mid_call(p2, hs2, dis, b2r, g2r, be2r, W3)
    p3 = agg(hs3, src3d, dst3d, zH)
    final_emb, out = _post_call(p3, hs3, dis, b3r, maskf, batch2)
    return final_emb, out


# speedup vs baseline: 1.0193x; 1.0193x over previous
import functools

import jax
import jax.numpy as jnp
from jax import lax
from jax.experimental import pallas as pl
from jax.experimental.pallas import tpu as pltpu
from jax.experimental.pallas import tpu_sc as plsc

N = 10000
E = 320000
D = 128
H = 128
G = 128

NC = 2
NS = 16
NW = NC * NS

CH = 128
TOT_CHUNKS = E // CH
NB = 3
NLOOP = 27
ACC_ROWS = N
RPT_A = 624
RPT_B = N - RPT_A * (NS - 1)
EPS = 1e-5

_HIGH = jax.lax.Precision.HIGHEST


@functools.cache
def _agg_kernel():
    mesh = plsc.VectorSubcoreMesh(core_axis_name="c", subcore_axis_name="s")

    @functools.partial(
        pl.kernel,
        mesh=mesh,
        out_type=jax.ShapeDtypeStruct((NC * ACC_ROWS, H), jnp.float32),
        scratch_types=[
            pltpu.VMEM((1, CH), jnp.int32),
            pltpu.VMEM((1, CH), jnp.int32),
            pltpu.VMEM((1, CH), jnp.int32),
            pltpu.VMEM((1, CH), jnp.int32),
            pltpu.VMEM((1, CH), jnp.int32),
            pltpu.VMEM((1, CH), jnp.int32),
            pltpu.VMEM((CH, H), jnp.float32),
            pltpu.VMEM((CH, H), jnp.float32),
            pltpu.VMEM((CH, H), jnp.float32),
            pltpu.VMEM_SHARED((ACC_ROWS, H), jnp.float32),
            pltpu.SemaphoreType.DMA,
            pltpu.SemaphoreType.DMA,
            pltpu.SemaphoreType.DMA,
            pltpu.SemaphoreType.DMA,
            pltpu.SemaphoreType.DMA,
            pltpu.SemaphoreType.DMA,
            pltpu.SemaphoreType.DMA,
            pltpu.SemaphoreType.DMA,
            pltpu.SemaphoreType.DMA,
        ],
    )
    def agg(hs_hbm, src_hbm, dst_hbm, zeros_hbm, out_hbm,
            sb0, sb1, sb2, db0, db1, db2, r0, r1, r2, acc_sh,
            g0, g1, g2, s0, s1, s2, i0, i1, i2):
        c = lax.axis_index("c")
        s = lax.axis_index("s")
        wid = s * NC + c
        cbase = (wid * TOT_CHUNKS) // NW
        cend = ((wid + 1) * TOT_CHUNKS) // NW
        sb = (sb0, sb1, sb2)
        db = (db0, db1, db2)
        rows = (r0, r1, r2)
        gsem = (g0, g1, g2)
        ssem = (s0, s1, s2)
        isem = (i0, i1, i2)

        for b in range(NB):
            pltpu.async_copy(src_hbm.at[cbase + b], sb[b], isem[b])
            pltpu.async_copy(dst_hbm.at[cbase + b], db[b], isem[b])
        for b in range(NB):
            pltpu.make_async_copy(src_hbm.at[0], sb[b], isem[b]).wait()
            pltpu.async_copy(hs_hbm.at[sb[b].at[0]], rows[b], gsem[b])

        @pl.when(s < NS - 1)
        def _():
            pltpu.sync_copy(zeros_hbm.at[pl.ds(0, RPT_A)],
                            acc_sh.at[pl.ds(s * RPT_A, RPT_A)])

        @pl.when(s == NS - 1)
        def _():
            pltpu.sync_copy(zeros_hbm,
                            acc_sh.at[pl.ds((NS - 1) * RPT_A, RPT_B)])

        plsc.subcore_barrier()

        def step(i, carry):
            for b in range(NB):
                gj = cbase + NB * i + b
                have = gj < cend
                nxt = gj + NB < cend

                @pl.when(have)
                def _():
                    pltpu.make_async_copy(hs_hbm.at[pl.ds(0, CH)],
                                          rows[b], gsem[b]).wait()
                    pltpu.make_async_copy(dst_hbm.at[0], db[b],
                                          isem[b]).wait()
                    pltpu.async_copy(rows[b], acc_sh.at[db[b].at[0]],
                                     ssem[b], add=True)

                @pl.when(nxt)
                def _():
                    pltpu.async_copy(src_hbm.at[gj + NB], sb[b], isem[b])

                @pl.when(have)
                def _():
                    pltpu.make_async_copy(rows[b], acc_sh.at[pl.ds(0, CH)],
                                          ssem[b]).wait()

                @pl.when(nxt)
                def _():
                    pltpu.async_copy(dst_hbm.at[gj + NB], db[b], isem[b])
                    pltpu.make_async_copy(src_hbm.at[0], sb[b],
                                          isem[b]).wait()
                    pltpu.async_copy(hs_hbm.at[sb[b].at[0]], rows[b], gsem[b])
            return carry

        lax.fori_loop(0, NLOOP, step, 0)
        plsc.subcore_barrier()

        @pl.when(s < NS - 1)
        def _():
            pltpu.sync_copy(acc_sh.at[pl.ds(s * RPT_A, RPT_A)],
                            out_hbm.at[pl.ds(c * ACC_ROWS + s * RPT_A, RPT_A)])

        @pl.when(s == NS - 1)
        def _():
            pltpu.sync_copy(
                acc_sh.at[pl.ds((NS - 1) * RPT_A, RPT_B)],
                out_hbm.at[pl.ds(c * ACC_ROWS + (NS - 1) * RPT_A, RPT_B)])

    return agg


CHE = 1280
NHC = E // CHE
HI = 80


def _pre_body(x_ref, w_ref, dst_ref, dis_ref, hs_ref):
    def body(j, acc):
        d2 = dst_ref[pl.ds(j, 1), :]
        hi2 = lax.shift_right_logical(d2, 7)
        lo2 = lax.bitwise_and(d2, 127)
        ohhi = (lax.broadcasted_iota(jnp.int32, (HI, CHE), 0) == hi2
                ).astype(jnp.float32)
        ohloT = (lax.broadcasted_iota(jnp.int32, (128, CHE), 0) == lo2
                 ).astype(jnp.float32)
        part = lax.dot_general(ohhi, ohloT, (((1,), (1,)), ((), ())),
                               preferred_element_type=jnp.float32)
        return acc + part

    deg = lax.fori_loop(0, NHC, body, jnp.zeros((HI, 128), jnp.float32))
    dis_grid = lax.rsqrt(deg + 1.0)
    ident = (lax.broadcasted_iota(jnp.int32, (128, 128), 0)
             == lax.broadcasted_iota(jnp.int32, (128, 128), 1)
             ).astype(jnp.float32)
    grid_t = lax.dot_general(ident, dis_grid, (((1,), (1,)), ((), ())),
                             precision=_HIGH,
                             preferred_element_type=jnp.float32)
    dis = jnp.concatenate(
        [grid_t[:, hi:hi + 1] for hi in range(HI)], axis=0)[0:N]
    h = lax.dot_general(x_ref[...], w_ref[...], (((1,), (0,)), ((), ())),
                        precision=_HIGH, preferred_element_type=jnp.float32)
    dis_ref[...] = dis
    hs_ref[...] = h * dis


_pre_call = pl.pallas_call(
    _pre_body,
    out_shape=(jax.ShapeDtypeStruct((N, 1), jnp.float32),
               jax.ShapeDtypeStruct((N, H), jnp.float32)),
)


def _mid_body(p_ref, hs_ref, dis_ref, b_ref, g_ref, be_ref, w_ref, out_ref):
    dis = dis_ref[...]
    agg = p_ref[0:N, :] + p_ref[ACC_ROWS:ACC_ROWS + N, :] + hs_ref[0:N, :]
    conv = agg * dis + b_ref[...]
    m = jnp.mean(conv, axis=0, keepdims=True)
    v = jnp.mean((conv - m) ** 2, axis=0, keepdims=True)
    y = (conv - m) * lax.rsqrt(v + EPS) * g_ref[...] + be_ref[...]
    y = jnp.maximum(y, 0.0)
    hn = lax.dot_general(y, w_ref[...], (((1,), (0,)), ((), ())),
                         precision=_HIGH, preferred_element_type=jnp.float32)
    out_ref[...] = hn * dis


_mid_call = pl.pallas_call(
    _mid_body,
    out_shape=jax.ShapeDtypeStruct((N, H), jnp.float32),
)


def _post_body(p_ref, hs_ref, dis_ref, b_ref, mask_ref, batch_ref,
               final_ref, out_ref):
    dis = dis_ref[...]
    agg = p_ref[0:N, :] + p_ref[ACC_ROWS:ACC_ROWS + N, :] + hs_ref[0:N, :]
    out = agg * dis + b_ref[...]
    pos = out * mask_ref[...]
    gids = lax.broadcasted_iota(jnp.int32, (G, N), 0)
    oh = (batch_ref[...] == gids).astype(jnp.float32)
    sums = lax.dot_general(oh, pos, (((1,), (0,)), ((), ())),
                           precision=_HIGH, preferred_element_type=jnp.float32)
    cnt = jnp.sum(oh, axis=1, keepdims=True)
    final_ref[...] = sums / jnp.maximum(cnt, 1.0)
    out_ref[...] = out


_post_call = pl.pallas_call(
    _post_body,
    out_shape=(jax.ShapeDtypeStruct((G, H), jnp.float32),
               jax.ShapeDtypeStruct((N, H), jnp.float32)),
)


def kernel(feature_matrix, edge_index, positive_doc_mask, batch,
           W1, b1, g1, be1, W2, b2, g2, be2, W3, b3):
    src = edge_index[0]
    dst = edge_index[1]
    src3d = src.reshape(TOT_CHUNKS, 1, CH)
    dst3d = dst.reshape(TOT_CHUNKS, 1, CH)

    zH = jnp.zeros((RPT_B, H), jnp.float32)
    maskf = positive_doc_mask.astype(jnp.float32)
    batch2 = batch.reshape(1, N)
    b1r, g1r, be1r = b1.reshape(1, H), g1.reshape(1, H), be1.reshape(1, H)
    b2r, g2r, be2r = b2.reshape(1, H), g2.reshape(1, H), be2.reshape(1, H)
    b3r = b3.reshape(1, H)

    agg = _agg_kernel()
    dis, hs1 = _pre_call(feature_matrix, W1, dst.reshape(NHC, CHE))
    p1 = agg(hs1, src3d, dst3d, zH)
    hs2 = _mid_call(p1, hs1, dis, b1r, g1r, be1r, W2)
    p2 = agg(hs2, src3d, dst3d, zH)
    hs3 = _mid_call(p2, hs2, dis, b2r, g2r, be2r, W3)
    p3 = agg(hs3, src3d, dst3d, zH)
    final_emb, out = _post_call(p3, hs3, dis, b3r, maskf, batch2)
    return final_emb, out
